# Initial kernel scaffold; baseline (speedup 1.0000x reference)
import jax
import jax.numpy as jnp
from jax.experimental import pallas as pl


def kernel(query, key, value, Wq_g, Wk_g):
    raise NotImplementedError("write your pallas kernel here")


# speedup vs baseline: 1.3716x; 1.3716x over previous
import numpy as np
import jax
import jax.numpy as jnp
from jax.experimental import pallas as pl
from jax.experimental.pallas import tpu as pltpu

_H = 32
_KVH = 8
_D = 128
_B = 128
_G = 64
_NREP = _H // _KVH
_TAU = 0.5


def _gate_scores_kernel(q_ref, k_ref, wq_ref, wk_ref, s_ref):
    T = q_ref.shape[0]
    nb = T // _B
    qb = q_ref[...].reshape(nb, _B, _D).mean(axis=1)
    kb = k_ref[...].reshape(nb, _B, _D).mean(axis=1)
    gq = jnp.dot(qb, wq_ref[...], preferred_element_type=jnp.float32)
    gk = jnp.dot(kb, wk_ref[...], preferred_element_type=jnp.float32)
    s_ref[0] = jnp.dot(gq, gk.T, preferred_element_type=jnp.float32)


def _flash_kernel(idx_ref, cnt_ref, q_ref, k_ref, v_ref, o_ref):
    h = pl.program_id(0)
    i = pl.program_id(1)
    scale = 1.0 / np.sqrt(_D)
    q = q_ref[...] * jnp.float32(scale)
    n = cnt_ref[h, i]
    row = jax.lax.broadcasted_iota(jnp.int32, (_B, _B), 0)
    col = jax.lax.broadcasted_iota(jnp.int32, (_B, _B), 1)

    def body(jj, carry):
        m_prev, l_prev, acc = carry
        j = idx_ref[h, i, jj]
        kblk = k_ref[pl.ds(j * _B, _B), :]
        s = jnp.dot(q, kblk.T, preferred_element_type=jnp.float32)
        keep = (j * _B + col) <= (i * _B + row)
        s = jnp.where(keep, s, -1e9)
        m_cur = jnp.max(s, axis=1, keepdims=True)
        m_new = jnp.maximum(m_prev, m_cur)
        p = jnp.exp(s - m_new)
        alpha = jnp.exp(m_prev - m_new)
        l_new = l_prev * alpha + jnp.sum(p, axis=1, keepdims=True)
        vblk = v_ref[pl.ds(j * _B, _B), :]
        acc_new = acc * alpha + jnp.dot(p, vblk, preferred_element_type=jnp.float32)
        return m_new, l_new, acc_new

    m0 = jnp.full((_B, 1), -1e30, jnp.float32)
    l0 = jnp.zeros((_B, 1), jnp.float32)
    acc0 = jnp.zeros((_B, _D), jnp.float32)
    _, l, acc = jax.lax.fori_loop(0, n, body, (m0, l0, acc0))
    o_ref[...] = acc / l


def kernel(query, key, value, Wq_g, Wk_g):
    T = query.shape[0]
    nb = T // _B

    scores = pl.pallas_call(
        _gate_scores_kernel,
        grid=(_H,),
        in_specs=[
            pl.BlockSpec((T, _D), lambda h: (0, h)),
            pl.BlockSpec((T, _D), lambda h: (0, h // _NREP)),
            pl.BlockSpec((_D, _G), lambda h: (0, 0)),
            pl.BlockSpec((_D, _G), lambda h: (0, 0)),
        ],
        out_specs=pl.BlockSpec((1, nb, nb), lambda h: (h, 0, 0)),
        out_shape=jax.ShapeDtypeStruct((_H, nb, nb), jnp.float32),
    )(query, key, Wq_g, Wk_g)

    gate = jax.nn.sigmoid(scores / jnp.sqrt(_G))
    iota = jnp.arange(nb)
    hard = (gate > _TAU) | (iota[:, None] == iota[None, :])
    act = hard & (iota[None, :, None] >= iota[None, None, :])
    cnt = act.sum(-1).astype(jnp.int32)
    sort_key = jnp.where(act, iota[None, None, :], nb)
    idx = jnp.argsort(sort_key, axis=-1).astype(jnp.int32)

    grid_spec = pltpu.PrefetchScalarGridSpec(
        num_scalar_prefetch=2,
        grid=(_H, nb),
        in_specs=[
            pl.BlockSpec((_B, _D), lambda h, i, idx_r, cnt_r: (i, h)),
            pl.BlockSpec((T, _D), lambda h, i, idx_r, cnt_r: (0, h // _NREP)),
            pl.BlockSpec((T, _D), lambda h, i, idx_r, cnt_r: (0, h // _NREP)),
        ],
        out_specs=pl.BlockSpec((_B, _D), lambda h, i, idx_r, cnt_r: (i, h)),
    )
    out = pl.pallas_call(
        _flash_kernel,
        grid_spec=grid_spec,
        out_shape=jax.ShapeDtypeStruct((T, _H * _D), jnp.float32),
    )(idx, cnt, query, key, value)
    return out


# speedup vs baseline: 1.5873x; 1.1572x over previous
import numpy as np
import jax
import jax.numpy as jnp
from jax.experimental import pallas as pl
from jax.experimental.pallas import tpu as pltpu

_H = 32
_KVH = 8
_D = 128
_B = 128
_G = 64
_NREP = _H // _KVH
_TAU = 0.5
_QT = 256
_CK = 512
_NEG = -1e9


def _gate_scores_kernel(q_ref, k_ref, wq_ref, wk_ref, s_ref):
    T = q_ref.shape[0]
    nb = T // _B
    qb = q_ref[...].reshape(nb, _B, _D).mean(axis=1)
    kb = k_ref[...].reshape(nb, _B, _D).mean(axis=1)
    gq = jnp.dot(qb, wq_ref[...], preferred_element_type=jnp.float32)
    gk = jnp.dot(kb, wk_ref[...], preferred_element_type=jnp.float32)
    s_ref[0] = jnp.dot(gq, gk.T, preferred_element_type=jnp.float32)


def _flash_kernel(bias_ref, q_ref, k_ref, v_ref, o_ref, m_scr, l_scr, acc_scr):
    i = pl.program_id(1)
    jc = pl.program_id(2)
    ncj = pl.num_programs(2)
    nrows = _QT // _B

    @pl.when(jc == 0)
    def _init():
        m_scr[...] = jnp.full((_QT, 1), -1e30, jnp.float32)
        l_scr[...] = jnp.zeros((_QT, 1), jnp.float32)
        acc_scr[...] = jnp.zeros((_QT, _D), jnp.float32)

    needed = (i * _QT + _QT - 1) // _CK + 1

    @pl.when(jc < needed)
    def _compute():
        scale = jnp.float32(1.0 / np.sqrt(_D))
        q = q_ref[...] * scale
        s = jnp.dot(q, k_ref[...].T, preferred_element_type=jnp.float32)
        bias = bias_ref[0].reshape(nrows, 1, _CK)
        s = (s.reshape(nrows, _B, _CK) + bias).reshape(_QT, _CK)
        row = jax.lax.broadcasted_iota(jnp.int32, (_QT, _CK), 0) + i * _QT
        col = jax.lax.broadcasted_iota(jnp.int32, (_QT, _CK), 1) + jc * _CK
        s = jnp.where(col <= row, s, _NEG)
        m_prev = m_scr[...]
        m_cur = jnp.max(s, axis=1, keepdims=True)
        m_new = jnp.maximum(m_prev, m_cur)
        alpha = jnp.exp(m_prev - m_new)
        p = jnp.exp(s - m_new)
        m_scr[...] = m_new
        l_scr[...] = l_scr[...] * alpha + jnp.sum(p, axis=1, keepdims=True)
        acc_scr[...] = acc_scr[...] * alpha + jnp.dot(
            p, v_ref[...], preferred_element_type=jnp.float32)

    @pl.when(jc == ncj - 1)
    def _finish():
        o_ref[...] = acc_scr[...] / l_scr[...]


def kernel(query, key, value, Wq_g, Wk_g):
    T = query.shape[0]
    nb = T // _B

    scores = pl.pallas_call(
        _gate_scores_kernel,
        grid=(_H,),
        in_specs=[
            pl.BlockSpec((T, _D), lambda h: (0, h)),
            pl.BlockSpec((T, _D), lambda h: (0, h // _NREP)),
            pl.BlockSpec((_D, _G), lambda h: (0, 0)),
            pl.BlockSpec((_D, _G), lambda h: (0, 0)),
        ],
        out_specs=pl.BlockSpec((1, nb, nb), lambda h: (h, 0, 0)),
        out_shape=jax.ShapeDtypeStruct((_H, nb, nb), jnp.float32),
    )(query, key, Wq_g, Wk_g)

    gate = jax.nn.sigmoid(scores / jnp.sqrt(_G))
    iota = jnp.arange(nb)
    hard = (gate > _TAU) | (iota[:, None] == iota[None, :])
    bias = jnp.where(hard, 0.0, _NEG).astype(jnp.float32)
    bias_tok = jnp.repeat(bias, _B, axis=2).reshape(_H, nb, 1, T)

    nqt = T // _QT
    ncj = T // _CK
    out = pl.pallas_call(
        _flash_kernel,
        grid=(_H, nqt, ncj),
        in_specs=[
            pl.BlockSpec((1, _QT // _B, 1, _CK),
                         lambda h, i, jc: (h, i, 0, jc)),
            pl.BlockSpec((_QT, _D), lambda h, i, jc: (i, h)),
            pl.BlockSpec((_CK, _D), lambda h, i, jc: (jc, h // _NREP)),
            pl.BlockSpec((_CK, _D), lambda h, i, jc: (jc, h // _NREP)),
        ],
        out_specs=pl.BlockSpec((_QT, _D), lambda h, i, jc: (i, h)),
        out_shape=jax.ShapeDtypeStruct((T, _H * _D), jnp.float32),
        scratch_shapes=[
            pltpu.VMEM((_QT, 1), jnp.float32),
            pltpu.VMEM((_QT, 1), jnp.float32),
            pltpu.VMEM((_QT, _D), jnp.float32),
        ],
        compiler_params=pltpu.CompilerParams(
            dimension_semantics=("parallel", "arbitrary", "arbitrary")),
    )(bias_tok, query, key, value)
    return out
